# baseline (speedup 1.0000x reference)
import functools

import jax
import jax.numpy as jnp
from jax import lax
from jax.experimental import pallas as pl
from jax.experimental.pallas import tpu as pltpu
from jax.experimental.pallas import tpu_sc as plsc

B, N, D = 16, 50000, 128
M = 2048
NC, NS, L = 2, 16, 16
NW = NC * NS
RPW = (B * M) // NW
GCHUNK = 128
WCHUNK = 256
NG = WCHUNK // GCHUNK
NCH = RPW // WCHUNK
NBUF = 3


@functools.partial(
    pl.kernel,
    mesh=plsc.VectorSubcoreMesh(core_axis_name="c", subcore_axis_name="s"),
    out_type=jax.ShapeDtypeStruct((B * M, D), jnp.float32),
    scratch_types=[
        pltpu.VMEM((RPW,), jnp.int32),
        pltpu.VMEM((NBUF, WCHUNK, D), jnp.float32),
        *([pltpu.SemaphoreType.DMA] * (2 * NBUF)),
    ],
)
def _gather_sc(data_hbm, idx_hbm, out_hbm, idx_v, bufs, *sems):
    gsem, wsem = sems[:NBUF], sems[NBUF:]
    wid = lax.axis_index("s") * NC + lax.axis_index("c")
    base = wid * RPW
    batch = base // M
    half = base % M
    off = batch * N

    pltpu.sync_copy(idx_hbm.at[batch, pl.ds(half, RPW)], idx_v)
    for i in range(RPW // L):
        sl = pl.ds(i * L, L)
        idx_v[sl] = idx_v[sl] + off

    def start_gathers(j):
        b = j % NBUF
        return [
            pltpu.async_copy(
                data_hbm.at[idx_v.at[pl.ds(j * WCHUNK + g * GCHUNK, GCHUNK)]],
                bufs.at[b, pl.ds(g * GCHUNK, GCHUNK)],
                gsem[b],
            )
            for g in range(NG)
        ]

    gd, wd = {}, {}
    for j in range(min(NBUF, NCH)):
        gd[j] = start_gathers(j)
    for j in range(NCH):
        b = j % NBUF
        for g in gd[j]:
            g.wait()
        wd[j] = pltpu.async_copy(
            bufs.at[b], out_hbm.at[pl.ds(base + j * WCHUNK, WCHUNK)], wsem[b]
        )
        if j + NBUF < NCH:
            wd[j].wait()
            gd[j + NBUF] = start_gathers(j + NBUF)
    for j in range(max(0, NCH - NBUF), NCH):
        wd[j].wait()


def kernel(data, indices):
    data_flat = data.reshape(B * N, D)
    out = _gather_sc(data_flat, indices.astype(jnp.int32))
    return out.reshape(B, M, D)


# speedup vs baseline: 1.0381x; 1.0381x over previous
import functools

import jax
import jax.numpy as jnp
from jax import lax
from jax.experimental import pallas as pl
from jax.experimental.pallas import tpu as pltpu
from jax.experimental.pallas import tpu_sc as plsc

B, N, D = 16, 50000, 128
M = 2048
NC, NS, L = 2, 16, 16
NW = NC * NS
RPW = (B * M) // NW
GCHUNK = 128
WCHUNK = 128
NG = WCHUNK // GCHUNK
NCH = RPW // WCHUNK
NBUF = 7


@functools.partial(
    pl.kernel,
    mesh=plsc.VectorSubcoreMesh(core_axis_name="c", subcore_axis_name="s"),
    out_type=jax.ShapeDtypeStruct((B * M, D), jnp.float32),
    scratch_types=[
        pltpu.VMEM((RPW,), jnp.int32),
        pltpu.VMEM((NBUF, WCHUNK, D), jnp.float32),
        *([pltpu.SemaphoreType.DMA] * (2 * NBUF)),
    ],
)
def _gather_sc(data_hbm, idx_hbm, out_hbm, idx_v, bufs, *sems):
    gsem, wsem = sems[:NBUF], sems[NBUF:]
    wid = lax.axis_index("s") * NC + lax.axis_index("c")
    base = wid * RPW
    batch = base // M
    half = base % M
    off = batch * N

    pltpu.sync_copy(idx_hbm.at[batch, pl.ds(half, RPW)], idx_v)
    for i in range(RPW // L):
        sl = pl.ds(i * L, L)
        idx_v[sl] = idx_v[sl] + off

    def start_gathers(j):
        b = j % NBUF
        return [
            pltpu.async_copy(
                data_hbm.at[idx_v.at[pl.ds(j * WCHUNK + g * GCHUNK, GCHUNK)]],
                bufs.at[b, pl.ds(g * GCHUNK, GCHUNK)],
                gsem[b],
            )
            for g in range(NG)
        ]

    gd, wd = {}, {}
    for j in range(min(NBUF, NCH)):
        gd[j] = start_gathers(j)
    for j in range(NCH):
        b = j % NBUF
        for g in gd[j]:
            g.wait()
        wd[j] = pltpu.async_copy(
            bufs.at[b], out_hbm.at[pl.ds(base + j * WCHUNK, WCHUNK)], wsem[b]
        )
        if j + NBUF < NCH:
            wd[j].wait()
            gd[j + NBUF] = start_gathers(j + NBUF)
    for j in range(max(0, NCH - NBUF), NCH):
        wd[j].wait()


def kernel(data, indices):
    data_flat = data.reshape(B * N, D)
    out = _gather_sc(data_flat, indices.astype(jnp.int32))
    return out.reshape(B, M, D)
